# Initial kernel scaffold; baseline (speedup 1.0000x reference)
import jax
import jax.numpy as jnp
from jax.experimental import pallas as pl


def kernel(atom_in_fea, nbr_fea, self_fea_idx, nbr_fea_idx, W, b, bn1_g, bn1_b, bn2_g, bn2_b):
    raise NotImplementedError("write your pallas kernel here")


# speedup vs baseline: 1.7193x; 1.7193x over previous
import functools

import jax
import jax.numpy as jnp
from jax import lax
from jax.experimental import pallas as pl
from jax.experimental.pallas import tpu as pltpu
from jax.experimental.pallas import tpu_sc as plsc

N_NODES = 10000
N_EDGES = 160000
F = 256
NBR = 16
OUT = 512
EPS = 1e-5

NC = 2
NS = 16
NW = NC * NS

SB = 128
GB = 112
CH_W = N_EDGES // NW
NFULL_W = CH_W // GB
TAIL_W = CH_W - NFULL_W * GB
CH_T = N_EDGES // NS
NFULL_T = CH_T // SB
TAIL_T = CH_T - NFULL_T * SB
CHALF = F // NC
DR = 624

EBLK = 2000
NBLK = 2000


def _proj_body(a_ref, ws_ref, wn_ref, ps_ref, pn_ref):
    a = a_ref[...]
    ps_ref[...] = jnp.dot(a, ws_ref[...], preferred_element_type=jnp.float32)
    pn_ref[...] = jnp.dot(a, wn_ref[...], preferred_element_type=jnp.float32)


def _project(atom, ws, wn):
    return pl.pallas_call(
        _proj_body,
        grid=(N_NODES // NBLK,),
        in_specs=[
            pl.BlockSpec((NBLK, F), lambda i: (i, 0)),
            pl.BlockSpec((F, OUT), lambda i: (0, 0)),
            pl.BlockSpec((F, OUT), lambda i: (0, 0)),
        ],
        out_specs=[
            pl.BlockSpec((NBLK, OUT), lambda i: (i, 0)),
            pl.BlockSpec((NBLK, OUT), lambda i: (i, 0)),
        ],
        out_shape=[jax.ShapeDtypeStruct((N_NODES, OUT), jnp.float32)] * 2,
    )(atom, ws, wn)


def _gather_body(ps_hbm, pn_hbm, sidx_hbm, nidx_hbm, xp_hbm,
                 sidx_v, nidx_v, bufa, bufb, sem1, sem2):
    wid = lax.axis_index("s") * NC + lax.axis_index("c")
    base = wid * CH_W
    pltpu.sync_copy(sidx_hbm.at[pl.ds(base, CH_W)], sidx_v)
    pltpu.sync_copy(nidx_hbm.at[pl.ds(base, CH_W)], nidx_v)

    def step(off, nrows):
        e0 = base + off
        da = bufa.at[pl.ds(0, nrows)]
        db = bufb.at[pl.ds(0, nrows)]
        ca = pltpu.async_copy(ps_hbm.at[sidx_v.at[pl.ds(off, nrows)]], da, sem1)
        cb = pltpu.async_copy(pn_hbm.at[nidx_v.at[pl.ds(off, nrows)]], db, sem2)
        ca.wait()
        cb.wait()

        def rbody(r, carry):
            for j in range(OUT // 16):
                sl = pl.ds(j * 16, 16)
                bufa[r, sl] = bufa[r, sl] + bufb[r, sl]
            return carry

        lax.fori_loop(0, nrows, rbody, 0)
        pltpu.sync_copy(da, xp_hbm.at[pl.ds(e0, nrows)])

    def body(i, carry):
        step(i * GB, GB)
        return carry

    lax.fori_loop(0, NFULL_W, body, 0)
    step(NFULL_W * GB, TAIL_W)


@functools.cache
def _gather_edges_call():
    return pl.kernel(
        _gather_body,
        out_type=jax.ShapeDtypeStruct((N_EDGES, OUT), jnp.float32),
        mesh=plsc.VectorSubcoreMesh(core_axis_name="c", subcore_axis_name="s"),
        scratch_types=[
            pltpu.VMEM((CH_W,), jnp.int32),
            pltpu.VMEM((CH_W,), jnp.int32),
            pltpu.VMEM((GB, OUT), jnp.float32),
            pltpu.VMEM((GB, OUT), jnp.float32),
            pltpu.SemaphoreType.DMA,
            pltpu.SemaphoreType.DMA,
        ],
    )


def _stats1_body(xp_ref, nbr_ref, we_ref, b_ref, st_ref):
    @pl.when(pl.program_id(0) == 0)
    def _init():
        st_ref[...] = jnp.zeros_like(st_ref)

    x = (xp_ref[...]
         + jnp.dot(nbr_ref[...], we_ref[...], preferred_element_type=jnp.float32)
         + b_ref[...])
    st_ref[...] += jnp.concatenate(
        [jnp.sum(x, axis=0, keepdims=True),
         jnp.sum(x * x, axis=0, keepdims=True)], axis=0)


def _stats1(xp, nbr_fea, we, b2d):
    return pl.pallas_call(
        _stats1_body,
        grid=(N_EDGES // EBLK,),
        in_specs=[
            pl.BlockSpec((EBLK, OUT), lambda i: (i, 0)),
            pl.BlockSpec((EBLK, NBR), lambda i: (i, 0)),
            pl.BlockSpec((NBR, OUT), lambda i: (0, 0)),
            pl.BlockSpec((1, OUT), lambda i: (0, 0)),
        ],
        out_specs=pl.BlockSpec((2, OUT), lambda i: (0, 0)),
        out_shape=jax.ShapeDtypeStruct((2, OUT), jnp.float32),
    )(xp, nbr_fea, we, b2d)


def _msg_body(xp_ref, nbr_ref, we_ref, b_ref, st_ref, g1_ref, b1_ref, msg_ref):
    x = (xp_ref[...]
         + jnp.dot(nbr_ref[...], we_ref[...], preferred_element_type=jnp.float32)
         + b_ref[...])
    mu = st_ref[0:1, :] * (1.0 / N_EDGES)
    var = st_ref[1:2, :] * (1.0 / N_EDGES) - mu * mu
    xh = (x - mu) * lax.rsqrt(var + EPS) * g1_ref[...] + b1_ref[...]
    filt = jax.nn.sigmoid(xh[:, :F])
    core = jax.nn.softplus(xh[:, F:])
    msg_ref[...] = filt * core


def _messages(xp, nbr_fea, we, b2d, st, g1, b1):
    return pl.pallas_call(
        _msg_body,
        grid=(N_EDGES // EBLK,),
        in_specs=[
            pl.BlockSpec((EBLK, OUT), lambda i: (i, 0)),
            pl.BlockSpec((EBLK, NBR), lambda i: (i, 0)),
            pl.BlockSpec((NBR, OUT), lambda i: (0, 0)),
            pl.BlockSpec((1, OUT), lambda i: (0, 0)),
            pl.BlockSpec((2, OUT), lambda i: (0, 0)),
            pl.BlockSpec((1, OUT), lambda i: (0, 0)),
            pl.BlockSpec((1, OUT), lambda i: (0, 0)),
        ],
        out_specs=pl.BlockSpec((EBLK, F), lambda i: (i, 0)),
        out_shape=jax.ShapeDtypeStruct((N_EDGES, F), jnp.float32),
    )(xp, nbr_fea, we, b2d, st, g1, b1)


def _segsum_body(msg_hbm, sidx_hbm, zero_hbm, pooled_hbm,
                 shared, idx_full, idx_tail, buf, sem):
    cid = lax.axis_index("c")
    sid = lax.axis_index("s")
    c0 = cid * CHALF
    n0 = sid * DR
    pltpu.sync_copy(zero_hbm, shared.at[pl.ds(n0, DR)])

    @pl.when(sid == NS - 1)
    def _zero_rest():
        pltpu.sync_copy(zero_hbm.at[pl.ds(0, N_NODES - NS * DR)],
                        shared.at[pl.ds(NS * DR, N_NODES - NS * DR)])

    plsc.subcore_barrier()
    base = sid * CH_T

    def body(i, carry):
        e0 = base + i * SB
        pltpu.sync_copy(sidx_hbm.at[pl.ds(e0, SB)], idx_full.at[0])
        pltpu.async_copy(
            msg_hbm.at[pl.ds(e0, SB), pl.ds(c0, CHALF)], buf, sem).wait()
        pltpu.sync_copy(buf, shared.at[idx_full.at[0]], add=True)
        return carry

    lax.fori_loop(0, NFULL_T, body, 0)
    e0 = base + NFULL_T * SB
    pltpu.sync_copy(sidx_hbm.at[pl.ds(e0, TAIL_T)], idx_tail.at[0])
    pltpu.async_copy(
        msg_hbm.at[pl.ds(e0, TAIL_T), pl.ds(c0, CHALF)],
        buf.at[pl.ds(0, TAIL_T)], sem).wait()
    pltpu.sync_copy(buf.at[pl.ds(0, TAIL_T)],
                    shared.at[idx_tail.at[0]], add=True)

    plsc.subcore_barrier()
    pltpu.sync_copy(shared.at[pl.ds(n0, DR)],
                    pooled_hbm.at[pl.ds(n0, DR), pl.ds(c0, CHALF)])

    @pl.when(sid == NS - 1)
    def _drain_rest():
        nr = N_NODES - NS * DR
        pltpu.sync_copy(shared.at[pl.ds(NS * DR, nr)],
                        pooled_hbm.at[pl.ds(NS * DR, nr), pl.ds(c0, CHALF)])


@functools.cache
def _segment_sum_call():
    return pl.kernel(
        _segsum_body,
        out_type=jax.ShapeDtypeStruct((N_NODES, F), jnp.float32),
        mesh=plsc.VectorSubcoreMesh(core_axis_name="c", subcore_axis_name="s"),
        scratch_types=[
            pltpu.VMEM_SHARED((N_NODES, CHALF), jnp.float32),
            pltpu.VMEM((1, SB), jnp.int32),
            pltpu.VMEM((1, TAIL_T), jnp.int32),
            pltpu.VMEM((SB, CHALF), jnp.float32),
            pltpu.SemaphoreType.DMA,
        ],
    )


def _stats2_body(p_ref, st_ref):
    @pl.when(pl.program_id(0) == 0)
    def _init():
        st_ref[...] = jnp.zeros_like(st_ref)

    x = p_ref[...]
    st_ref[...] += jnp.concatenate(
        [jnp.sum(x, axis=0, keepdims=True),
         jnp.sum(x * x, axis=0, keepdims=True)], axis=0)


def _stats2(pooled):
    return pl.pallas_call(
        _stats2_body,
        grid=(N_NODES // NBLK,),
        in_specs=[pl.BlockSpec((NBLK, F), lambda i: (i, 0))],
        out_specs=pl.BlockSpec((2, F), lambda i: (0, 0)),
        out_shape=jax.ShapeDtypeStruct((2, F), jnp.float32),
    )(pooled)


def _final_body(p_ref, a_ref, st_ref, g2_ref, b2_ref, o_ref):
    mu = st_ref[0:1, :] * (1.0 / N_NODES)
    var = st_ref[1:2, :] * (1.0 / N_NODES) - mu * mu
    y = (p_ref[...] - mu) * lax.rsqrt(var + EPS) * g2_ref[...] + b2_ref[...]
    o_ref[...] = jax.nn.softplus(a_ref[...] + y)


def _finalize(pooled, atom, st2, g2, b2):
    return pl.pallas_call(
        _final_body,
        grid=(N_NODES // NBLK,),
        in_specs=[
            pl.BlockSpec((NBLK, F), lambda i: (i, 0)),
            pl.BlockSpec((NBLK, F), lambda i: (i, 0)),
            pl.BlockSpec((2, F), lambda i: (0, 0)),
            pl.BlockSpec((1, F), lambda i: (0, 0)),
            pl.BlockSpec((1, F), lambda i: (0, 0)),
        ],
        out_specs=pl.BlockSpec((NBLK, F), lambda i: (i, 0)),
        out_shape=jax.ShapeDtypeStruct((N_NODES, F), jnp.float32),
    )(pooled, atom, st2, g2, b2)


def kernel(atom_in_fea, nbr_fea, self_fea_idx, nbr_fea_idx, W, b,
           bn1_g, bn1_b, bn2_g, bn2_b):
    sidx = self_fea_idx.astype(jnp.int32)
    nidx = nbr_fea_idx.astype(jnp.int32)
    ws = W[:F]
    wn = W[F:2 * F]
    we = W[2 * F:]
    b2d = b.reshape(1, OUT)
    g1 = bn1_g.reshape(1, OUT)
    b1 = bn1_b.reshape(1, OUT)
    g2 = bn2_g.reshape(1, F)
    b2 = bn2_b.reshape(1, F)
    zero = jnp.zeros((DR, CHALF), jnp.float32)

    ps, pn = _project(atom_in_fea, ws, wn)
    xp = _gather_edges_call()(ps, pn, sidx, nidx)
    st1 = _stats1(xp, nbr_fea, we, b2d)
    msg = _messages(xp, nbr_fea, we, b2d, st1, g1, b1)
    pooled = _segment_sum_call()(msg, sidx, zero)
    st2 = _stats2(pooled)
    return _finalize(pooled, atom_in_fea, st2, g2, b2)


# speedup vs baseline: 1.9503x; 1.1343x over previous
import functools

import jax
import jax.numpy as jnp
from jax import lax
from jax.experimental import pallas as pl
from jax.experimental.pallas import tpu as pltpu
from jax.experimental.pallas import tpu_sc as plsc

N_NODES = 10000
N_EDGES = 160000
F = 256
NBR = 16
OUT = 512
EPS = 1e-5

NC = 2
NS = 16
NW = NC * NS

SB = 128
GB = 40
CH_W = N_EDGES // NW
NSTEP_W = CH_W // GB
PAIRS_W = NSTEP_W // 2
CH_T = N_EDGES // NS
NFULL_T = CH_T // SB
TAIL_T = CH_T - NFULL_T * SB
CHALF = F // NC
DR = 624

EBLK = 2000
NBLK = 2000


def _proj_body(a_ref, ws_ref, wn_ref, ps_ref, pn_ref):
    a = a_ref[...]
    ps_ref[...] = jnp.dot(a, ws_ref[...], preferred_element_type=jnp.float32)
    pn_ref[...] = jnp.dot(a, wn_ref[...], preferred_element_type=jnp.float32)


def _project(atom, ws, wn):
    return pl.pallas_call(
        _proj_body,
        grid=(N_NODES // NBLK,),
        in_specs=[
            pl.BlockSpec((NBLK, F), lambda i: (i, 0)),
            pl.BlockSpec((F, OUT), lambda i: (0, 0)),
            pl.BlockSpec((F, OUT), lambda i: (0, 0)),
        ],
        out_specs=[
            pl.BlockSpec((NBLK, OUT), lambda i: (i, 0)),
            pl.BlockSpec((NBLK, OUT), lambda i: (i, 0)),
        ],
        out_shape=[jax.ShapeDtypeStruct((N_NODES, OUT), jnp.float32)] * 2,
    )(atom, ws, wn)


def _gather_body(ps_hbm, pn_hbm, sidx_hbm, nidx_hbm, xp_hbm,
                 sidx_v, nidx_v, a0, b0, a1, b1,
                 sga0, sgb0, sga1, sgb1, sst0, sst1):
    wid = lax.axis_index("s") * NC + lax.axis_index("c")
    base = wid * CH_W
    pltpu.sync_copy(sidx_hbm.at[pl.ds(base, CH_W)], sidx_v)
    pltpu.sync_copy(nidx_hbm.at[pl.ds(base, CH_W)], nidx_v)

    A = (a0, a1)
    B = (b0, b1)
    SGA = (sga0, sga1)
    SGB = (sgb0, sgb1)
    SST = (sst0, sst1)

    def issue_g(off, p):
        pltpu.async_copy(ps_hbm.at[sidx_v.at[pl.ds(off, GB)]], A[p], SGA[p])
        pltpu.async_copy(pn_hbm.at[nidx_v.at[pl.ds(off, GB)]], B[p], SGB[p])

    def wait_g(off, p):
        pltpu.make_async_copy(
            ps_hbm.at[sidx_v.at[pl.ds(off, GB)]], A[p], SGA[p]).wait()
        pltpu.make_async_copy(
            pn_hbm.at[nidx_v.at[pl.ds(off, GB)]], B[p], SGB[p]).wait()

    def wait_st(off, p):
        pltpu.make_async_copy(
            A[p], xp_hbm.at[pl.ds(base + off, GB)], SST[p]).wait()

    def do_step(i, q, guard=False, last=False):
        off = i * GB

        if guard:
            @pl.when(i > 0)
            def _wait_prev_store():
                wait_st(off - GB, 1 - q)
        else:
            wait_st(off - GB, 1 - q)

        if not last:
            issue_g(off + GB, 1 - q)
        wait_g(off, q)

        def rbody(r, carry):
            for j in range(OUT // 16):
                sl = pl.ds(j * 16, 16)
                A[q][r, sl] = A[q][r, sl] + B[q][r, sl]
            return carry

        lax.fori_loop(0, GB, rbody, 0)
        pltpu.async_copy(A[q], xp_hbm.at[pl.ds(base + off, GB)], SST[q])

    issue_g(0, 0)

    def pair(k, carry):
        do_step(2 * k, 0, guard=True)
        do_step(2 * k + 1, 1)
        return carry

    lax.fori_loop(0, PAIRS_W, pair, 0)
    do_step(NSTEP_W - 1, 0, last=True)
    wait_st((NSTEP_W - 1) * GB, 0)


@functools.cache
def _gather_edges_call():
    return pl.kernel(
        _gather_body,
        out_type=jax.ShapeDtypeStruct((N_EDGES, OUT), jnp.float32),
        mesh=plsc.VectorSubcoreMesh(core_axis_name="c", subcore_axis_name="s"),
        scratch_types=[
            pltpu.VMEM((CH_W,), jnp.int32),
            pltpu.VMEM((CH_W,), jnp.int32),
            pltpu.VMEM((GB, OUT), jnp.float32),
            pltpu.VMEM((GB, OUT), jnp.float32),
            pltpu.VMEM((GB, OUT), jnp.float32),
            pltpu.VMEM((GB, OUT), jnp.float32),
            pltpu.SemaphoreType.DMA,
            pltpu.SemaphoreType.DMA,
            pltpu.SemaphoreType.DMA,
            pltpu.SemaphoreType.DMA,
            pltpu.SemaphoreType.DMA,
            pltpu.SemaphoreType.DMA,
        ],
    )


def _stats1_body(xp_ref, nbr_ref, we_ref, b_ref, st_ref):
    @pl.when(pl.program_id(0) == 0)
    def _init():
        st_ref[...] = jnp.zeros_like(st_ref)

    x = (xp_ref[...]
         + jnp.dot(nbr_ref[...], we_ref[...], preferred_element_type=jnp.float32)
         + b_ref[...])
    st_ref[...] += jnp.concatenate(
        [jnp.sum(x, axis=0, keepdims=True),
         jnp.sum(x * x, axis=0, keepdims=True)], axis=0)


def _stats1(xp, nbr_fea, we, b2d):
    return pl.pallas_call(
        _stats1_body,
        grid=(N_EDGES // EBLK,),
        in_specs=[
            pl.BlockSpec((EBLK, OUT), lambda i: (i, 0)),
            pl.BlockSpec((EBLK, NBR), lambda i: (i, 0)),
            pl.BlockSpec((NBR, OUT), lambda i: (0, 0)),
            pl.BlockSpec((1, OUT), lambda i: (0, 0)),
        ],
        out_specs=pl.BlockSpec((2, OUT), lambda i: (0, 0)),
        out_shape=jax.ShapeDtypeStruct((2, OUT), jnp.float32),
    )(xp, nbr_fea, we, b2d)


def _msg_body(xp_ref, nbr_ref, we_ref, b_ref, st_ref, g1_ref, b1_ref, msg_ref):
    x = (xp_ref[...]
         + jnp.dot(nbr_ref[...], we_ref[...], preferred_element_type=jnp.float32)
         + b_ref[...])
    mu = st_ref[0:1, :] * (1.0 / N_EDGES)
    var = st_ref[1:2, :] * (1.0 / N_EDGES) - mu * mu
    xh = (x - mu) * lax.rsqrt(var + EPS) * g1_ref[...] + b1_ref[...]
    filt = jax.nn.sigmoid(xh[:, :F])
    core = jax.nn.softplus(xh[:, F:])
    msg_ref[...] = filt * core


def _messages(xp, nbr_fea, we, b2d, st, g1, b1):
    return pl.pallas_call(
        _msg_body,
        grid=(N_EDGES // EBLK,),
        in_specs=[
            pl.BlockSpec((EBLK, OUT), lambda i: (i, 0)),
            pl.BlockSpec((EBLK, NBR), lambda i: (i, 0)),
            pl.BlockSpec((NBR, OUT), lambda i: (0, 0)),
            pl.BlockSpec((1, OUT), lambda i: (0, 0)),
            pl.BlockSpec((2, OUT), lambda i: (0, 0)),
            pl.BlockSpec((1, OUT), lambda i: (0, 0)),
            pl.BlockSpec((1, OUT), lambda i: (0, 0)),
        ],
        out_specs=pl.BlockSpec((EBLK, F), lambda i: (i, 0)),
        out_shape=jax.ShapeDtypeStruct((N_EDGES, F), jnp.float32),
    )(xp, nbr_fea, we, b2d, st, g1, b1)


def _segsum_body(msg_hbm, sidx_hbm, zero_hbm, pooled_hbm,
                 shared, idx_full, idx_tail, buf, sem):
    cid = lax.axis_index("c")
    sid = lax.axis_index("s")
    c0 = cid * CHALF
    n0 = sid * DR
    pltpu.sync_copy(zero_hbm, shared.at[pl.ds(n0, DR)])

    @pl.when(sid == NS - 1)
    def _zero_rest():
        pltpu.sync_copy(zero_hbm.at[pl.ds(0, N_NODES - NS * DR)],
                        shared.at[pl.ds(NS * DR, N_NODES - NS * DR)])

    plsc.subcore_barrier()
    base = sid * CH_T

    def body(i, carry):
        e0 = base + i * SB
        pltpu.sync_copy(sidx_hbm.at[pl.ds(e0, SB)], idx_full.at[0])
        pltpu.async_copy(
            msg_hbm.at[pl.ds(e0, SB), pl.ds(c0, CHALF)], buf, sem).wait()
        pltpu.sync_copy(buf, shared.at[idx_full.at[0]], add=True)
        return carry

    lax.fori_loop(0, NFULL_T, body, 0)
    e0 = base + NFULL_T * SB
    pltpu.sync_copy(sidx_hbm.at[pl.ds(e0, TAIL_T)], idx_tail.at[0])
    pltpu.async_copy(
        msg_hbm.at[pl.ds(e0, TAIL_T), pl.ds(c0, CHALF)],
        buf.at[pl.ds(0, TAIL_T)], sem).wait()
    pltpu.sync_copy(buf.at[pl.ds(0, TAIL_T)],
                    shared.at[idx_tail.at[0]], add=True)

    plsc.subcore_barrier()
    pltpu.sync_copy(shared.at[pl.ds(n0, DR)],
                    pooled_hbm.at[pl.ds(n0, DR), pl.ds(c0, CHALF)])

    @pl.when(sid == NS - 1)
    def _drain_rest():
        nr = N_NODES - NS * DR
        pltpu.sync_copy(shared.at[pl.ds(NS * DR, nr)],
                        pooled_hbm.at[pl.ds(NS * DR, nr), pl.ds(c0, CHALF)])


@functools.cache
def _segment_sum_call():
    return pl.kernel(
        _segsum_body,
        out_type=jax.ShapeDtypeStruct((N_NODES, F), jnp.float32),
        mesh=plsc.VectorSubcoreMesh(core_axis_name="c", subcore_axis_name="s"),
        scratch_types=[
            pltpu.VMEM_SHARED((N_NODES, CHALF), jnp.float32),
            pltpu.VMEM((1, SB), jnp.int32),
            pltpu.VMEM((1, TAIL_T), jnp.int32),
            pltpu.VMEM((SB, CHALF), jnp.float32),
            pltpu.SemaphoreType.DMA,
        ],
    )


def _stats2_body(p_ref, st_ref):
    @pl.when(pl.program_id(0) == 0)
    def _init():
        st_ref[...] = jnp.zeros_like(st_ref)

    x = p_ref[...]
    st_ref[...] += jnp.concatenate(
        [jnp.sum(x, axis=0, keepdims=True),
         jnp.sum(x * x, axis=0, keepdims=True)], axis=0)


def _stats2(pooled):
    return pl.pallas_call(
        _stats2_body,
        grid=(N_NODES // NBLK,),
        in_specs=[pl.BlockSpec((NBLK, F), lambda i: (i, 0))],
        out_specs=pl.BlockSpec((2, F), lambda i: (0, 0)),
        out_shape=jax.ShapeDtypeStruct((2, F), jnp.float32),
    )(pooled)


def _final_body(p_ref, a_ref, st_ref, g2_ref, b2_ref, o_ref):
    mu = st_ref[0:1, :] * (1.0 / N_NODES)
    var = st_ref[1:2, :] * (1.0 / N_NODES) - mu * mu
    y = (p_ref[...] - mu) * lax.rsqrt(var + EPS) * g2_ref[...] + b2_ref[...]
    o_ref[...] = jax.nn.softplus(a_ref[...] + y)


def _finalize(pooled, atom, st2, g2, b2):
    return pl.pallas_call(
        _final_body,
        grid=(N_NODES // NBLK,),
        in_specs=[
            pl.BlockSpec((NBLK, F), lambda i: (i, 0)),
            pl.BlockSpec((NBLK, F), lambda i: (i, 0)),
            pl.BlockSpec((2, F), lambda i: (0, 0)),
            pl.BlockSpec((1, F), lambda i: (0, 0)),
            pl.BlockSpec((1, F), lambda i: (0, 0)),
        ],
        out_specs=pl.BlockSpec((NBLK, F), lambda i: (i, 0)),
        out_shape=jax.ShapeDtypeStruct((N_NODES, F), jnp.float32),
    )(pooled, atom, st2, g2, b2)


def kernel(atom_in_fea, nbr_fea, self_fea_idx, nbr_fea_idx, W, b,
           bn1_g, bn1_b, bn2_g, bn2_b):
    sidx = self_fea_idx.astype(jnp.int32)
    nidx = nbr_fea_idx.astype(jnp.int32)
    ws = W[:F]
    wn = W[F:2 * F]
    we = W[2 * F:]
    b2d = b.reshape(1, OUT)
    g1 = bn1_g.reshape(1, OUT)
    b1 = bn1_b.reshape(1, OUT)
    g2 = bn2_g.reshape(1, F)
    b2 = bn2_b.reshape(1, F)
    zero = jnp.zeros((DR, CHALF), jnp.float32)

    ps, pn = _project(atom_in_fea, ws, wn)
    xp = _gather_edges_call()(ps, pn, sidx, nidx)
    st1 = _stats1(xp, nbr_fea, we, b2d)
    msg = _messages(xp, nbr_fea, we, b2d, st1, g1, b1)
    pooled = _segment_sum_call()(msg, sidx, zero)
    st2 = _stats2(pooled)
    return _finalize(pooled, atom_in_fea, st2, g2, b2)


# speedup vs baseline: 2.1208x; 1.0874x over previous
import functools

import jax
import jax.numpy as jnp
from jax import lax
from jax.experimental import pallas as pl
from jax.experimental.pallas import tpu as pltpu
from jax.experimental.pallas import tpu_sc as plsc

N_NODES = 10000
N_EDGES = 160000
F = 256
NBR = 16
OUT = 512
EPS = 1e-5

NC = 2
NS = 16
NW = NC * NS

SB = 128
GB = 40
CH_W = N_EDGES // NW
NSTEP_W = CH_W // GB
PAIRS_W = NSTEP_W // 2
CH_T = N_EDGES // NS
MB = SB
NFULL_T = CH_T // MB
TAIL_T = CH_T - NFULL_T * MB
CHALF = F // NC
DR = 624

EBLK = 2000
NBLK = 2000


def _proj_body(a_ref, ws_ref, wn_ref, ps_ref, pn_ref):
    a = a_ref[...]
    ps_ref[...] = jnp.dot(a, ws_ref[...], preferred_element_type=jnp.float32)
    pn_ref[...] = jnp.dot(a, wn_ref[...], preferred_element_type=jnp.float32)


def _project(atom, ws, wn):
    return pl.pallas_call(
        _proj_body,
        grid=(N_NODES // NBLK,),
        in_specs=[
            pl.BlockSpec((NBLK, F), lambda i: (i, 0)),
            pl.BlockSpec((F, OUT), lambda i: (0, 0)),
            pl.BlockSpec((F, OUT), lambda i: (0, 0)),
        ],
        out_specs=[
            pl.BlockSpec((NBLK, OUT), lambda i: (i, 0)),
            pl.BlockSpec((NBLK, OUT), lambda i: (i, 0)),
        ],
        out_shape=[jax.ShapeDtypeStruct((N_NODES, OUT), jnp.float32)] * 2,
    )(atom, ws, wn)


def _gather_body(ps_hbm, pn_hbm, sidx_hbm, nidx_hbm, xp_hbm,
                 sidx_v, nidx_v, a0, b0, a1, b1,
                 sga0, sgb0, sga1, sgb1, sst0, sst1):
    wid = lax.axis_index("s") * NC + lax.axis_index("c")
    base = wid * CH_W
    pltpu.sync_copy(sidx_hbm.at[pl.ds(base, CH_W)], sidx_v)
    pltpu.sync_copy(nidx_hbm.at[pl.ds(base, CH_W)], nidx_v)

    A = (a0, a1)
    B = (b0, b1)
    SGA = (sga0, sga1)
    SGB = (sgb0, sgb1)
    SST = (sst0, sst1)

    def issue_g(off, p):
        pltpu.async_copy(ps_hbm.at[sidx_v.at[pl.ds(off, GB)]], A[p], SGA[p])
        pltpu.async_copy(pn_hbm.at[nidx_v.at[pl.ds(off, GB)]], B[p], SGB[p])

    def wait_g(off, p):
        pltpu.make_async_copy(
            ps_hbm.at[sidx_v.at[pl.ds(off, GB)]], A[p], SGA[p]).wait()
        pltpu.make_async_copy(
            pn_hbm.at[nidx_v.at[pl.ds(off, GB)]], B[p], SGB[p]).wait()

    def wait_st(off, p):
        pltpu.make_async_copy(
            A[p], xp_hbm.at[pl.ds(base + off, GB)], SST[p]).wait()

    def do_step(i, q, guard=False, last=False):
        off = i * GB

        if guard:
            @pl.when(i > 0)
            def _wait_prev_store():
                wait_st(off - GB, 1 - q)
        else:
            wait_st(off - GB, 1 - q)

        if not last:
            issue_g(off + GB, 1 - q)
        wait_g(off, q)

        def rbody(r, carry):
            for j in range(OUT // 16):
                sl = pl.ds(j * 16, 16)
                A[q][r, sl] = A[q][r, sl] + B[q][r, sl]
            return carry

        lax.fori_loop(0, GB, rbody, 0)
        pltpu.async_copy(A[q], xp_hbm.at[pl.ds(base + off, GB)], SST[q])

    issue_g(0, 0)

    def pair(k, carry):
        do_step(2 * k, 0, guard=True)
        do_step(2 * k + 1, 1)
        return carry

    lax.fori_loop(0, PAIRS_W, pair, 0)
    do_step(NSTEP_W - 1, 0, last=True)
    wait_st((NSTEP_W - 1) * GB, 0)


@functools.cache
def _gather_edges_call():
    return pl.kernel(
        _gather_body,
        out_type=jax.ShapeDtypeStruct((N_EDGES, OUT), jnp.float32),
        mesh=plsc.VectorSubcoreMesh(core_axis_name="c", subcore_axis_name="s"),
        scratch_types=[
            pltpu.VMEM((CH_W,), jnp.int32),
            pltpu.VMEM((CH_W,), jnp.int32),
            pltpu.VMEM((GB, OUT), jnp.float32),
            pltpu.VMEM((GB, OUT), jnp.float32),
            pltpu.VMEM((GB, OUT), jnp.float32),
            pltpu.VMEM((GB, OUT), jnp.float32),
            pltpu.SemaphoreType.DMA,
            pltpu.SemaphoreType.DMA,
            pltpu.SemaphoreType.DMA,
            pltpu.SemaphoreType.DMA,
            pltpu.SemaphoreType.DMA,
            pltpu.SemaphoreType.DMA,
        ],
    )


def _stats1_body(xp_ref, nbr_ref, we_ref, b_ref, st_ref):
    @pl.when(pl.program_id(0) == 0)
    def _init():
        st_ref[...] = jnp.zeros_like(st_ref)

    x = (xp_ref[...]
         + jnp.dot(nbr_ref[...], we_ref[...], preferred_element_type=jnp.float32)
         + b_ref[...])
    st_ref[...] += jnp.concatenate(
        [jnp.sum(x, axis=0, keepdims=True),
         jnp.sum(x * x, axis=0, keepdims=True)], axis=0)


def _stats1(xp, nbr_fea, we, b2d):
    return pl.pallas_call(
        _stats1_body,
        grid=(N_EDGES // EBLK,),
        in_specs=[
            pl.BlockSpec((EBLK, OUT), lambda i: (i, 0)),
            pl.BlockSpec((EBLK, NBR), lambda i: (i, 0)),
            pl.BlockSpec((NBR, OUT), lambda i: (0, 0)),
            pl.BlockSpec((1, OUT), lambda i: (0, 0)),
        ],
        out_specs=pl.BlockSpec((2, OUT), lambda i: (0, 0)),
        out_shape=jax.ShapeDtypeStruct((2, OUT), jnp.float32),
    )(xp, nbr_fea, we, b2d)


def _msg_body(xp_ref, nbr_ref, we_ref, b_ref, st_ref, g1_ref, b1_ref, msg_ref):
    x = (xp_ref[...]
         + jnp.dot(nbr_ref[...], we_ref[...], preferred_element_type=jnp.float32)
         + b_ref[...])
    mu = st_ref[0:1, :] * (1.0 / N_EDGES)
    var = st_ref[1:2, :] * (1.0 / N_EDGES) - mu * mu
    xh = (x - mu) * lax.rsqrt(var + EPS) * g1_ref[...] + b1_ref[...]
    filt = jax.nn.sigmoid(xh[:, :F])
    core = jax.nn.softplus(xh[:, F:])
    msg_ref[...] = filt * core


def _messages(xp, nbr_fea, we, b2d, st, g1, b1):
    return pl.pallas_call(
        _msg_body,
        grid=(N_EDGES // EBLK,),
        in_specs=[
            pl.BlockSpec((EBLK, OUT), lambda i: (i, 0)),
            pl.BlockSpec((EBLK, NBR), lambda i: (i, 0)),
            pl.BlockSpec((NBR, OUT), lambda i: (0, 0)),
            pl.BlockSpec((1, OUT), lambda i: (0, 0)),
            pl.BlockSpec((2, OUT), lambda i: (0, 0)),
            pl.BlockSpec((1, OUT), lambda i: (0, 0)),
            pl.BlockSpec((1, OUT), lambda i: (0, 0)),
        ],
        out_specs=pl.BlockSpec((EBLK, F), lambda i: (i, 0)),
        out_shape=jax.ShapeDtypeStruct((N_EDGES, F), jnp.float32),
    )(xp, nbr_fea, we, b2d, st, g1, b1)


def _segsum_body(msg_hbm, sidx_hbm, zero_hbm, pooled_hbm,
                 shared, m0, m1, i0, i1, it, sm0, sm1, si0, si1, sit):
    cid = lax.axis_index("c")
    sid = lax.axis_index("s")
    c0 = cid * CHALF
    n0 = sid * DR
    base = sid * CH_T
    MBUF = (m0, m1)
    IBUF = (i0, i1)
    SM = (sm0, sm1)
    SI = (si0, si1)

    def issue_ld(off, p):
        e0 = base + off
        pltpu.async_copy(
            msg_hbm.at[pl.ds(e0, MB), pl.ds(c0, CHALF)], MBUF[p], SM[p])
        pltpu.async_copy(sidx_hbm.at[pl.ds(e0, SB)], IBUF[p].at[0], SI[p])

    def wait_ld(off, p):
        e0 = base + off
        pltpu.make_async_copy(
            msg_hbm.at[pl.ds(e0, MB), pl.ds(c0, CHALF)], MBUF[p], SM[p]).wait()
        pltpu.make_async_copy(
            sidx_hbm.at[pl.ds(e0, SB)], IBUF[p].at[0], SI[p]).wait()

    issue_ld(0, 0)
    pltpu.sync_copy(zero_hbm, shared.at[pl.ds(n0, DR)])

    @pl.when(sid == NS - 1)
    def _zero_rest():
        pltpu.sync_copy(zero_hbm.at[pl.ds(0, N_NODES - NS * DR)],
                        shared.at[pl.ds(NS * DR, N_NODES - NS * DR)])

    plsc.subcore_barrier()

    def do_step(i, q, last=False):
        off = i * MB
        if not last:
            issue_ld(off + MB, 1 - q)
        wait_ld(off, q)
        pltpu.sync_copy(MBUF[q], shared.at[IBUF[q].at[0]], add=True)

    def pair(k, carry):
        do_step(2 * k, 0)
        do_step(2 * k + 1, 1)
        return carry

    lax.fori_loop(0, NFULL_T // 2 - 1, pair, 0)
    do_step(NFULL_T - 2, 0)
    do_step(NFULL_T - 1, 1, last=True)

    e0 = base + NFULL_T * MB
    pltpu.sync_copy(sidx_hbm.at[pl.ds(e0, TAIL_T)], it.at[0])
    pltpu.async_copy(
        msg_hbm.at[pl.ds(e0, TAIL_T), pl.ds(c0, CHALF)],
        m1.at[pl.ds(0, TAIL_T)], sm1).wait()
    pltpu.sync_copy(m1.at[pl.ds(0, TAIL_T)], shared.at[it.at[0]], add=True)

    plsc.subcore_barrier()
    pltpu.sync_copy(shared.at[pl.ds(n0, DR)],
                    pooled_hbm.at[pl.ds(n0, DR), pl.ds(c0, CHALF)])

    @pl.when(sid == NS - 1)
    def _drain_rest():
        nr = N_NODES - NS * DR
        pltpu.sync_copy(shared.at[pl.ds(NS * DR, nr)],
                        pooled_hbm.at[pl.ds(NS * DR, nr), pl.ds(c0, CHALF)])


@functools.cache
def _segment_sum_call():
    return pl.kernel(
        _segsum_body,
        out_type=jax.ShapeDtypeStruct((N_NODES, F), jnp.float32),
        mesh=plsc.VectorSubcoreMesh(core_axis_name="c", subcore_axis_name="s"),
        scratch_types=[
            pltpu.VMEM_SHARED((N_NODES, CHALF), jnp.float32),
            pltpu.VMEM((MB, CHALF), jnp.float32),
            pltpu.VMEM((MB, CHALF), jnp.float32),
            pltpu.VMEM((1, SB), jnp.int32),
            pltpu.VMEM((1, SB), jnp.int32),
            pltpu.VMEM((1, TAIL_T), jnp.int32),
            pltpu.SemaphoreType.DMA,
            pltpu.SemaphoreType.DMA,
            pltpu.SemaphoreType.DMA,
            pltpu.SemaphoreType.DMA,
            pltpu.SemaphoreType.DMA,
        ],
    )


def _stats2_body(p_ref, st_ref):
    @pl.when(pl.program_id(0) == 0)
    def _init():
        st_ref[...] = jnp.zeros_like(st_ref)

    x = p_ref[...]
    st_ref[...] += jnp.concatenate(
        [jnp.sum(x, axis=0, keepdims=True),
         jnp.sum(x * x, axis=0, keepdims=True)], axis=0)


def _stats2(pooled):
    return pl.pallas_call(
        _stats2_body,
        grid=(N_NODES // NBLK,),
        in_specs=[pl.BlockSpec((NBLK, F), lambda i: (i, 0))],
        out_specs=pl.BlockSpec((2, F), lambda i: (0, 0)),
        out_shape=jax.ShapeDtypeStruct((2, F), jnp.float32),
    )(pooled)


def _final_body(p_ref, a_ref, st_ref, g2_ref, b2_ref, o_ref):
    mu = st_ref[0:1, :] * (1.0 / N_NODES)
    var = st_ref[1:2, :] * (1.0 / N_NODES) - mu * mu
    y = (p_ref[...] - mu) * lax.rsqrt(var + EPS) * g2_ref[...] + b2_ref[...]
    o_ref[...] = jax.nn.softplus(a_ref[...] + y)


def _finalize(pooled, atom, st2, g2, b2):
    return pl.pallas_call(
        _final_body,
        grid=(N_NODES // NBLK,),
        in_specs=[
            pl.BlockSpec((NBLK, F), lambda i: (i, 0)),
            pl.BlockSpec((NBLK, F), lambda i: (i, 0)),
            pl.BlockSpec((2, F), lambda i: (0, 0)),
            pl.BlockSpec((1, F), lambda i: (0, 0)),
            pl.BlockSpec((1, F), lambda i: (0, 0)),
        ],
        out_specs=pl.BlockSpec((NBLK, F), lambda i: (i, 0)),
        out_shape=jax.ShapeDtypeStruct((N_NODES, F), jnp.float32),
    )(pooled, atom, st2, g2, b2)


def kernel(atom_in_fea, nbr_fea, self_fea_idx, nbr_fea_idx, W, b,
           bn1_g, bn1_b, bn2_g, bn2_b):
    sidx = self_fea_idx.astype(jnp.int32)
    nidx = nbr_fea_idx.astype(jnp.int32)
    ws = W[:F]
    wn = W[F:2 * F]
    we = W[2 * F:]
    b2d = b.reshape(1, OUT)
    g1 = bn1_g.reshape(1, OUT)
    b1 = bn1_b.reshape(1, OUT)
    g2 = bn2_g.reshape(1, F)
    b2 = bn2_b.reshape(1, F)
    zero = jnp.zeros((DR, CHALF), jnp.float32)

    ps, pn = _project(atom_in_fea, ws, wn)
    xp = _gather_edges_call()(ps, pn, sidx, nidx)
    st1 = _stats1(xp, nbr_fea, we, b2d)
    msg = _messages(xp, nbr_fea, we, b2d, st1, g1, b1)
    pooled = _segment_sum_call()(msg, sidx, zero)
    st2 = _stats2(pooled)
    return _finalize(pooled, atom_in_fea, st2, g2, b2)
